# baseline (speedup 1.0000x reference)
import functools

import jax
import jax.numpy as jnp
from jax import lax
from jax.experimental import pallas as pl
from jax.experimental.pallas import tpu as pltpu
from jax.experimental.pallas import tpu_sc as plsc

N_NODES = 10000
N_EDGES = 160000
IN_DIM = 256
HID_DIM = 256
OUT_DIM = 128

NC = 2
NS = 16
DEG_W = 16
CHUNK = 128
E_PAD = 163840
ROWS2D = E_PAD // CHUNK
ACC_ROWS = 10112
SHARE = ACC_ROWS // NS
PAD_ROW = N_NODES
HALF = HID_DIM // 2

R_BLK = 400
N_RBLK = N_NODES // R_BLK


def _sc_mesh():
    return plsc.VectorSubcoreMesh(core_axis_name="c", subcore_axis_name="s")


def _degree_hist(dst3d, ones_chunk, zeros_share):
    rows_per_tile = dst3d.shape[1] // NS

    @functools.partial(
        pl.kernel,
        out_type=jax.ShapeDtypeStruct((NC, ACC_ROWS, DEG_W), jnp.float32),
        mesh=_sc_mesh(),
        compiler_params=pltpu.CompilerParams(use_tc_tiling_on_sc=False),
        scratch_types=[
            pltpu.VMEM((rows_per_tile, CHUNK), jnp.int32),
            pltpu.VMEM((CHUNK, DEG_W), jnp.float32),
            pltpu.VMEM_SHARED((ACC_ROWS, DEG_W), jnp.float32),
            pltpu.SemaphoreType.DMA,
        ],
    )
    def deg_kernel(dst_hbm, ones_hbm, zeros_hbm, out_hbm, idx_v, ones_v, acc, sem):
        cid = lax.axis_index("c")
        sid = lax.axis_index("s")
        pltpu.sync_copy(zeros_hbm, acc.at[pl.ds(sid * SHARE, SHARE)])
        pltpu.sync_copy(ones_hbm, ones_v)
        pltpu.sync_copy(dst_hbm.at[cid, pl.ds(sid * rows_per_tile, rows_per_tile)], idx_v)
        plsc.subcore_barrier()

        @pl.loop(0, rows_per_tile)
        def _(j):
            pltpu.sync_copy(ones_v, acc.at[idx_v.at[j]], add=True)

        plsc.subcore_barrier()
        pltpu.sync_copy(
            acc.at[pl.ds(sid * SHARE, SHARE)],
            out_hbm.at[cid, pl.ds(sid * SHARE, SHARE)],
        )

    return deg_kernel(dst3d, ones_chunk, zeros_share)


def _edge_aggregate(table, src3d, dst3d, d):
    rows_per_tile = src3d.shape[1] // NS
    idxb = min(rows_per_tile, 40)
    n_pass = rows_per_tile // idxb
    nbuf = 2 if d > 64 else 4

    @functools.partial(
        pl.kernel,
        out_type=jax.ShapeDtypeStruct((NC, ACC_ROWS, d), jnp.float32),
        mesh=_sc_mesh(),
        compiler_params=pltpu.CompilerParams(use_tc_tiling_on_sc=False),
        scratch_types=[
            pltpu.VMEM((idxb, CHUNK), jnp.int32),
            pltpu.VMEM((idxb, CHUNK), jnp.int32),
        ] + [pltpu.VMEM((CHUNK, d), jnp.float32)] * nbuf + [
            pltpu.VMEM_SHARED((ACC_ROWS, d), jnp.float32),
        ] + [pltpu.SemaphoreType.DMA] * nbuf,
    )
    def agg_kernel(tab_hbm, src_hbm, dst_hbm, out_hbm,
                   idxs, idxd, *rest):
        bufs = rest[:nbuf]
        acc = rest[nbuf]
        sems = rest[nbuf + 1: 2 * nbuf + 1]
        cid = lax.axis_index("c")
        sid = lax.axis_index("s")
        pltpu.sync_copy(
            tab_hbm.at[pl.ds(cid * ACC_ROWS + sid * SHARE, SHARE)],
            acc.at[pl.ds(sid * SHARE, SHARE)],
        )
        plsc.subcore_barrier()

        for p in range(n_pass):
            base = sid * rows_per_tile + p * idxb
            pltpu.sync_copy(src_hbm.at[cid, pl.ds(base, idxb)], idxs)
            pltpu.sync_copy(dst_hbm.at[cid, pl.ds(base, idxb)], idxd)

            for b in range(nbuf - 1):
                pltpu.async_copy(tab_hbm.at[idxs.at[b]], bufs[b], sems[b])

            @pl.loop(0, idxb, step=nbuf)
            def _(j):
                for b in range(nbuf):
                    nxt = j + b + nbuf - 1

                    @pl.when(nxt < idxb)
                    def _():
                        pltpu.async_copy(
                            tab_hbm.at[idxs.at[nxt]],
                            bufs[(b + nbuf - 1) % nbuf],
                            sems[(b + nbuf - 1) % nbuf],
                        )

                    pltpu.make_async_copy(
                        tab_hbm.at[idxs.at[0]], bufs[b], sems[b]).wait()
                    pltpu.sync_copy(bufs[b], acc.at[idxd.at[j + b]], add=True)

        plsc.subcore_barrier()
        pltpu.sync_copy(
            acc.at[pl.ds(sid * SHARE, SHARE)],
            out_hbm.at[cid, pl.ds(sid * SHARE, SHARE)],
        )

    return agg_kernel(table, src3d, dst3d)


def _deg_spec():
    return pl.BlockSpec((NC, R_BLK, DEG_W), lambda i: (0, i, 0))


def _dis_from(deg_ref):
    deg = deg_ref[0, :, 0] + deg_ref[1, :, 0] + 1.0
    return lax.rsqrt(deg)


def _mm1_scale(x, W1, degp):
    def body(x_ref, w_ref, d_ref, o_ref):
        dis = _dis_from(d_ref)
        h = jnp.dot(x_ref[...], w_ref[...], preferred_element_type=jnp.float32)
        h = h * dis[:, None]
        o_ref[0] = h[:, :HALF]
        o_ref[1] = h[:, HALF:]

    return pl.pallas_call(
        body,
        grid=(N_RBLK,),
        in_specs=[
            pl.BlockSpec((R_BLK, IN_DIM), lambda i: (i, 0)),
            pl.BlockSpec((IN_DIM, HID_DIM), lambda i: (0, 0)),
            _deg_spec(),
        ],
        out_specs=pl.BlockSpec((NC, R_BLK, HALF), lambda i: (0, i, 0)),
        out_shape=jax.ShapeDtypeStruct((NC, ACC_ROWS, HALF), jnp.float32),
    )(x, W1, degp)


def _combine_mm2(agg1, degp, W2, b1):
    def body(a_ref, d_ref, w_ref, b_ref, o_ref):
        dis = _dis_from(d_ref)
        z = jnp.concatenate([a_ref[0], a_ref[1]], axis=1)
        h1 = jnp.maximum(z * dis[:, None] + b_ref[...], 0.0)
        t = jnp.dot(h1, w_ref[...], preferred_element_type=jnp.float32)
        t = t * dis[:, None]
        o_ref[0] = t[:, : OUT_DIM // 2]
        o_ref[1] = t[:, OUT_DIM // 2:]

    return pl.pallas_call(
        body,
        grid=(N_RBLK,),
        in_specs=[
            pl.BlockSpec((NC, R_BLK, HALF), lambda i: (0, i, 0)),
            _deg_spec(),
            pl.BlockSpec((HID_DIM, OUT_DIM), lambda i: (0, 0)),
            pl.BlockSpec((1, HID_DIM), lambda i: (0, 0)),
        ],
        out_specs=pl.BlockSpec((NC, R_BLK, OUT_DIM // 2), lambda i: (0, i, 0)),
        out_shape=jax.ShapeDtypeStruct((NC, ACC_ROWS, OUT_DIM // 2), jnp.float32),
    )(agg1, degp, W2, b1)


def _final_softmax(agg2, degp, b2):
    def body(a_ref, d_ref, b_ref, o_ref):
        dis = _dis_from(d_ref)
        z = jnp.concatenate([a_ref[0], a_ref[1]], axis=1)
        o = z * dis[:, None] + b_ref[...]
        m = jnp.max(o, axis=1, keepdims=True)
        e = jnp.exp(o - m)
        s = jnp.sum(e, axis=1, keepdims=True)
        o_ref[...] = (o - m) - jnp.log(s)

    return pl.pallas_call(
        body,
        grid=(N_RBLK,),
        in_specs=[
            pl.BlockSpec((NC, R_BLK, OUT_DIM // 2), lambda i: (0, i, 0)),
            _deg_spec(),
            pl.BlockSpec((1, OUT_DIM), lambda i: (0, 0)),
        ],
        out_specs=pl.BlockSpec((R_BLK, OUT_DIM), lambda i: (i, 0)),
        out_shape=jax.ShapeDtypeStruct((N_NODES, OUT_DIM), jnp.float32),
    )(agg2, degp, b2)


def kernel(x, edge_index, W1, b1, W2, b2):
    src = edge_index[0].astype(jnp.int32)
    dst = edge_index[1].astype(jnp.int32)
    pad = E_PAD - N_EDGES
    srcp = jnp.concatenate([src, jnp.zeros((pad,), jnp.int32)])
    dstp = jnp.concatenate([dst, jnp.full((pad,), PAD_ROW, jnp.int32)])
    dstF3d = jnp.stack([dstp, dstp]).reshape(NC, ROWS2D, CHUNK)
    srcF3d = jnp.stack([srcp, srcp + ACC_ROWS]).reshape(NC, ROWS2D, CHUNK)
    dstE3d = dstp.reshape(NC, ROWS2D // NC, CHUNK)

    ones_chunk = jnp.ones((CHUNK, DEG_W), jnp.float32)
    zeros16 = jnp.zeros((SHARE, DEG_W), jnp.float32)

    degp = _degree_hist(dstE3d, ones_chunk, zeros16)
    h2 = _mm1_scale(x, W1, degp)
    agg1 = _edge_aggregate(h2.reshape(NC * ACC_ROWS, HALF),
                           srcF3d, dstF3d, HALF)
    t2 = _combine_mm2(agg1, degp, W2, b1.reshape(1, -1))
    agg2 = _edge_aggregate(t2.reshape(NC * ACC_ROWS, OUT_DIM // 2),
                           srcF3d, dstF3d, OUT_DIM // 2)
    return _final_softmax(agg2, degp, b2.reshape(1, -1))


# speedup vs baseline: 1.0134x; 1.0134x over previous
import functools

import jax
import jax.numpy as jnp
from jax import lax
from jax.experimental import pallas as pl
from jax.experimental.pallas import tpu as pltpu
from jax.experimental.pallas import tpu_sc as plsc

N_NODES = 10000
N_EDGES = 160000
IN_DIM = 256
HID_DIM = 256
OUT_DIM = 128

NC = 2
NS = 16
DEG_W = 16
CHUNK = 128
E_PAD = 163840
ROWS2D = E_PAD // CHUNK
ACC_ROWS = 10112
SHARE = ACC_ROWS // NS
PAD_ROW = N_NODES
HALF = HID_DIM // 2

R_BLK = 400
N_RBLK = N_NODES // R_BLK


def _sc_mesh():
    return plsc.VectorSubcoreMesh(core_axis_name="c", subcore_axis_name="s")


def _degree_hist(dst3d, ones_chunk, zeros_share):
    rows_per_tile = dst3d.shape[1] // NS

    @functools.partial(
        pl.kernel,
        out_type=jax.ShapeDtypeStruct((NC, ACC_ROWS, DEG_W), jnp.float32),
        mesh=_sc_mesh(),
        compiler_params=pltpu.CompilerParams(use_tc_tiling_on_sc=False),
        scratch_types=[
            pltpu.VMEM((rows_per_tile, CHUNK), jnp.int32),
            pltpu.VMEM((CHUNK, DEG_W), jnp.float32),
            pltpu.VMEM_SHARED((ACC_ROWS, DEG_W), jnp.float32),
            pltpu.SemaphoreType.DMA,
        ],
    )
    def deg_kernel(dst_hbm, ones_hbm, zeros_hbm, out_hbm, idx_v, ones_v, acc, sem):
        cid = lax.axis_index("c")
        sid = lax.axis_index("s")
        pltpu.sync_copy(zeros_hbm, acc.at[pl.ds(sid * SHARE, SHARE)])
        pltpu.sync_copy(ones_hbm, ones_v)
        pltpu.sync_copy(dst_hbm.at[cid, pl.ds(sid * rows_per_tile, rows_per_tile)], idx_v)
        plsc.subcore_barrier()

        @pl.loop(0, rows_per_tile)
        def _(j):
            pltpu.sync_copy(ones_v, acc.at[idx_v.at[j]], add=True)

        plsc.subcore_barrier()
        pltpu.sync_copy(
            acc.at[pl.ds(sid * SHARE, SHARE)],
            out_hbm.at[cid, pl.ds(sid * SHARE, SHARE)],
        )

    return deg_kernel(dst3d, ones_chunk, zeros_share)


def _edge_aggregate(table, src3d, dst3d, zeros_share, d):
    rows_per_tile = src3d.shape[1] // NS
    idxb = min(rows_per_tile, 40)
    n_pass = rows_per_tile // idxb
    nbuf = 2 if d > 64 else 4

    @functools.partial(
        pl.kernel,
        out_type=jax.ShapeDtypeStruct((NC, ACC_ROWS, d), jnp.float32),
        mesh=_sc_mesh(),
        compiler_params=pltpu.CompilerParams(use_tc_tiling_on_sc=False),
        scratch_types=[
            pltpu.VMEM((idxb, CHUNK), jnp.int32),
            pltpu.VMEM((idxb, CHUNK), jnp.int32),
        ] + [pltpu.VMEM((CHUNK, d), jnp.float32)] * nbuf + [
            pltpu.VMEM_SHARED((ACC_ROWS, d), jnp.float32),
        ] + [pltpu.SemaphoreType.DMA] * nbuf,
    )
    def agg_kernel(tab_hbm, src_hbm, dst_hbm, zeros_hbm, out_hbm,
                   idxs, idxd, *rest):
        bufs = rest[:nbuf]
        acc = rest[nbuf]
        sems = rest[nbuf + 1: 2 * nbuf + 1]
        cid = lax.axis_index("c")
        sid = lax.axis_index("s")
        pltpu.sync_copy(zeros_hbm, acc.at[pl.ds(sid * SHARE, SHARE)])
        plsc.subcore_barrier()

        for p in range(n_pass):
            base = sid * rows_per_tile + p * idxb
            pltpu.sync_copy(src_hbm.at[cid, pl.ds(base, idxb)], idxs)
            pltpu.sync_copy(dst_hbm.at[cid, pl.ds(base, idxb)], idxd)

            for b in range(nbuf - 1):
                pltpu.async_copy(tab_hbm.at[idxs.at[b]], bufs[b], sems[b])

            @pl.loop(0, idxb, step=nbuf)
            def _(j):
                for b in range(nbuf):
                    nxt = j + b + nbuf - 1

                    @pl.when(nxt < idxb)
                    def _():
                        pltpu.async_copy(
                            tab_hbm.at[idxs.at[nxt]],
                            bufs[(b + nbuf - 1) % nbuf],
                            sems[(b + nbuf - 1) % nbuf],
                        )

                    pltpu.make_async_copy(
                        tab_hbm.at[idxs.at[0]], bufs[b], sems[b]).wait()
                    pltpu.sync_copy(bufs[b], acc.at[idxd.at[j + b]], add=True)

        plsc.subcore_barrier()
        pltpu.sync_copy(
            acc.at[pl.ds(sid * SHARE, SHARE)],
            out_hbm.at[cid, pl.ds(sid * SHARE, SHARE)],
        )

    return agg_kernel(table, src3d, dst3d, zeros_share)


def _deg_spec():
    return pl.BlockSpec((NC, R_BLK, DEG_W), lambda i: (0, i, 0))


def _dis_from(deg_ref):
    deg = deg_ref[0, :, 0] + deg_ref[1, :, 0] + 1.0
    return lax.rsqrt(deg)


def _mm1_scale(x, W1, degp):
    def body(x_ref, w_ref, d_ref, o_ref):
        dis = _dis_from(d_ref)
        h = jnp.dot(x_ref[...], w_ref[...], preferred_element_type=jnp.float32)
        h = h * dis[:, None]
        o_ref[0] = h[:, :HALF]
        o_ref[1] = h[:, HALF:]

    return pl.pallas_call(
        body,
        grid=(N_RBLK,),
        in_specs=[
            pl.BlockSpec((R_BLK, IN_DIM), lambda i: (i, 0)),
            pl.BlockSpec((IN_DIM, HID_DIM), lambda i: (0, 0)),
            _deg_spec(),
        ],
        out_specs=pl.BlockSpec((NC, R_BLK, HALF), lambda i: (0, i, 0)),
        out_shape=jax.ShapeDtypeStruct((NC, N_NODES, HALF), jnp.float32),
    )(x, W1, degp)


def _combine_mm2(agg1, h2, degp, W2, b1):
    def body(a_ref, h_ref, d_ref, w_ref, b_ref, o_ref):
        dis = _dis_from(d_ref)
        z = jnp.concatenate([a_ref[0] + h_ref[0], a_ref[1] + h_ref[1]], axis=1)
        h1 = jnp.maximum(z * dis[:, None] + b_ref[...], 0.0)
        t = jnp.dot(h1, w_ref[...], preferred_element_type=jnp.float32)
        t = t * dis[:, None]
        o_ref[0] = t[:, : OUT_DIM // 2]
        o_ref[1] = t[:, OUT_DIM // 2:]

    half_spec = pl.BlockSpec((NC, R_BLK, HALF), lambda i: (0, i, 0))
    return pl.pallas_call(
        body,
        grid=(N_RBLK,),
        in_specs=[
            half_spec,
            half_spec,
            _deg_spec(),
            pl.BlockSpec((HID_DIM, OUT_DIM), lambda i: (0, 0)),
            pl.BlockSpec((1, HID_DIM), lambda i: (0, 0)),
        ],
        out_specs=pl.BlockSpec((NC, R_BLK, OUT_DIM // 2), lambda i: (0, i, 0)),
        out_shape=jax.ShapeDtypeStruct((NC, N_NODES, OUT_DIM // 2), jnp.float32),
    )(agg1, h2, degp, W2, b1)


def _final_softmax(agg2, t2, degp, b2):
    def body(a_ref, t_ref, d_ref, b_ref, o_ref):
        dis = _dis_from(d_ref)
        z = jnp.concatenate([a_ref[0] + t_ref[0], a_ref[1] + t_ref[1]], axis=1)
        o = z * dis[:, None] + b_ref[...]
        m = jnp.max(o, axis=1, keepdims=True)
        e = jnp.exp(o - m)
        s = jnp.sum(e, axis=1, keepdims=True)
        o_ref[...] = (o - m) - jnp.log(s)

    return pl.pallas_call(
        body,
        grid=(N_RBLK,),
        in_specs=[
            pl.BlockSpec((NC, R_BLK, OUT_DIM // 2), lambda i: (0, i, 0)),
            pl.BlockSpec((NC, R_BLK, OUT_DIM // 2), lambda i: (0, i, 0)),
            _deg_spec(),
            pl.BlockSpec((1, OUT_DIM), lambda i: (0, 0)),
        ],
        out_specs=pl.BlockSpec((R_BLK, OUT_DIM), lambda i: (i, 0)),
        out_shape=jax.ShapeDtypeStruct((N_NODES, OUT_DIM), jnp.float32),
    )(agg2, t2, degp, b2)


def kernel(x, edge_index, W1, b1, W2, b2):
    src = edge_index[0].astype(jnp.int32)
    dst = edge_index[1].astype(jnp.int32)
    pad = E_PAD - N_EDGES
    srcp = jnp.concatenate([src, jnp.zeros((pad,), jnp.int32)])
    dstp = jnp.concatenate([dst, jnp.full((pad,), PAD_ROW, jnp.int32)])
    dstF3d = jnp.stack([dstp, dstp]).reshape(NC, ROWS2D, CHUNK)
    srcF3d = jnp.stack([srcp, srcp + N_NODES]).reshape(NC, ROWS2D, CHUNK)
    dstE3d = dstp.reshape(NC, ROWS2D // NC, CHUNK)

    ones_chunk = jnp.ones((CHUNK, DEG_W), jnp.float32)
    zeros16 = jnp.zeros((SHARE, DEG_W), jnp.float32)
    zeros64 = jnp.zeros((SHARE, OUT_DIM // 2), jnp.float32)
    zeros128 = jnp.zeros((SHARE, HALF), jnp.float32)

    degp = _degree_hist(dstE3d, ones_chunk, zeros16)
    h2 = _mm1_scale(x, W1, degp)
    agg1 = _edge_aggregate(h2.reshape(NC * N_NODES, HALF),
                           srcF3d, dstF3d, zeros128, HALF)
    agg1 = agg1[:, :N_NODES, :]
    t2 = _combine_mm2(agg1, h2, degp, W2, b1.reshape(1, -1))
    agg2 = _edge_aggregate(t2.reshape(NC * N_NODES, OUT_DIM // 2),
                           srcF3d, dstF3d, zeros64, OUT_DIM // 2)
    agg2 = agg2[:, :N_NODES, :]
    return _final_softmax(agg2, t2, degp, b2.reshape(1, -1))
